# baseline (speedup 1.0000x reference)
import jax
import jax.numpy as jnp
from jax import lax
from jax.experimental import pallas as pl
from jax.experimental.pallas import tpu as pltpu
from jax.experimental.pallas import tpu_sc as plsc

N = 1048576
NC, NS, L = 2, 16, 16
NW = NC * NS
NWL = NW * L
SH = N // NW
SS = SH // L
WL = 512
BTC = 1024
NPAD = N + 8 * BTC
NB = NPAD // BTC
NBP = 1056
W3N = 2048
NW3 = NPAD // W3N


def _route(xv, yv, zv):
    cx = (jnp.clip((xv + 1.0) / 2.0, 0.0, 0.99) * 2.0).astype(jnp.int32)
    cy = (jnp.clip((yv + 1.0) / 2.0, 0.0, 0.99) * 2.0).astype(jnp.int32)
    cz = (jnp.clip((zv + 1.0) / 2.0, 0.0, 0.99) * 2.0).astype(jnp.int32)
    return cx + 2 * cy + 4 * cz


def _wid():
    return lax.axis_index("s") * NC + lax.axis_index("c")


def _p1_body(xt_hbm, yt_hbm, zt_hbm, hist_hbm, xw, yw, zw, cv8, sx, sy, sz):
    wid = _wid()
    base = wid * (SS * L)

    def win_loop(w, acc):
        start = base + w * (WL * L)
        cx_ = pltpu.async_copy(xt_hbm.at[pl.ds(start, WL * L)], xw, sx)
        cy_ = pltpu.async_copy(yt_hbm.at[pl.ds(start, WL * L)], yw, sy)
        cz_ = pltpu.async_copy(zt_hbm.at[pl.ds(start, WL * L)], zw, sz)
        cx_.wait(); cy_.wait(); cz_.wait()

        def step(i, acc):
            o = i * L
            b = _route(xw[pl.ds(o, L)], yw[pl.ds(o, L)], zw[pl.ds(o, L)])
            return tuple(acc[e] + jnp.where(b == e, 1, 0) for e in range(8))

        return lax.fori_loop(0, WL, step, acc)

    acc0 = tuple(jnp.zeros((L,), jnp.int32) for _ in range(8))
    acc = lax.fori_loop(0, SS // WL, win_loop, acc0)
    for e in range(8):
        cv8[e] = acc[e]
    pltpu.sync_copy(cv8, hist_hbm.at[wid])


def _p2_body(xt_hbm, yt_hbm, zt_hbm, wbase_hbm,
             si_hbm, xs_hbm, ys_hbm, zs_hbm,
             xw, yw, zw, piw, dstw, wrows, sx, sy, sz, s0, s1, s2, s3):
    wid = _wid()
    base = wid * (SS * L)
    ii = lax.iota(jnp.int32, L)
    pltpu.sync_copy(wbase_hbm.at[wid], wrows)
    cnts = tuple(wrows[e] for e in range(8))

    def win_loop(w, cnts):
        start = base + w * (WL * L)
        cx_ = pltpu.async_copy(xt_hbm.at[pl.ds(start, WL * L)], xw, sx)
        cy_ = pltpu.async_copy(yt_hbm.at[pl.ds(start, WL * L)], yw, sy)
        cz_ = pltpu.async_copy(zt_hbm.at[pl.ds(start, WL * L)], zw, sz)
        cx_.wait(); cy_.wait(); cz_.wait()
        pv0 = wid * SH + ii * SS + w * WL

        def step(i, cnts):
            o = i * L
            b = _route(xw[pl.ds(o, L)], yw[pl.ds(o, L)], zw[pl.ds(o, L)])
            b0 = (b & 1) == 1
            b1 = (b & 2) == 2
            b2 = (b & 4) == 4
            d01 = jnp.where(b0, cnts[1], cnts[0])
            d23 = jnp.where(b0, cnts[3], cnts[2])
            d45 = jnp.where(b0, cnts[5], cnts[4])
            d67 = jnp.where(b0, cnts[7], cnts[6])
            dlo = jnp.where(b1, d23, d01)
            dhi = jnp.where(b1, d67, d45)
            dstw[pl.ds(o, L)] = jnp.where(b2, dhi, dlo)
            piw[pl.ds(o, L)] = pv0 + i
            return tuple(cnts[e] + jnp.where(b == e, 1, 0) for e in range(8))

        cnts = lax.fori_loop(0, WL, step, cnts)
        cc0 = pltpu.async_copy(xw, xs_hbm.at[dstw], s0)
        cc1 = pltpu.async_copy(yw, ys_hbm.at[dstw], s1)
        cc2 = pltpu.async_copy(zw, zs_hbm.at[dstw], s2)
        cc3 = pltpu.async_copy(piw, si_hbm.at[dstw], s3)
        cc0.wait(); cc1.wait(); cc2.wait(); cc3.wait()
        return cnts

    lax.fori_loop(0, SS // WL, win_loop, cnts)


def _tc_body(emap_ref, xs_ref, ys_ref, zs_ref, wl_ref, w2t_ref, out_ref):
    e = emap_ref[0, 0, 0]
    wl = wl_ref[e]
    xr = xs_ref[0]
    yr = ys_ref[0]
    zr = zs_ref[0]
    h1 = jnp.maximum(
        wl[:, 0:1] * xr + wl[:, 1:2] * yr + wl[:, 2:3] * zr + wl[:, 3:4], 0.0)
    h2 = jnp.maximum(
        jnp.dot(w2t_ref[e], h1.astype(jnp.bfloat16),
                preferred_element_type=jnp.float32) + wl[:, 4:5], 0.0)
    y = jnp.sum(h2 * wl[:, 5:6], axis=0, keepdims=True) + wl[0:1, 6:7]
    out_ref[...] = y[None]


def _p3_body(ysort_hbm, si_hbm, vblk_hbm, yfull_hbm,
             yw, siw, dstw, vblkv, sem):
    wid = _wid()
    ii = lax.iota(jnp.int32, L)
    pltpu.sync_copy(vblk_hbm, vblkv)

    def win_loop(k, _):
        win = wid + NW * k

        @pl.when(win < NW3)
        def _():
            start = win * W3N
            pltpu.async_copy(ysort_hbm.at[pl.ds(start, W3N)], yw, sem).wait()
            pltpu.async_copy(si_hbm.at[pl.ds(start, W3N)], siw, sem).wait()
            blk0 = win * (W3N // BTC)
            vbv = vblkv[pl.ds(blk0, L)]
            vb0 = vbv[0]
            vb1 = vbv[1]

            def vec2(i, _):
                o = i * L
                rp = (o % BTC) + ii
                inblk1 = o >= BTC
                vb = jnp.where(inblk1, vb1, vb0)
                sv = siw[pl.ds(o, L)]
                dstw[pl.ds(o, L)] = jnp.where(rp < vb, sv, N + ii)
                return 0

            lax.fori_loop(0, W3N // L, vec2, 0)
            pltpu.async_copy(yw, yfull_hbm.at[dstw], sem).wait()

        return 0

    lax.fori_loop(0, (NW3 + NW - 1) // NW, win_loop, 0)


@jax.jit
def kernel(x, W1, b1, W2, b2, W3, b3, local_min, local_max):
    f32, i32 = jnp.float32, jnp.int32
    mesh = plsc.VectorSubcoreMesh(core_axis_name="c", subcore_axis_name="s")

    xt3 = x.reshape(NW, L, SS, 3).transpose(0, 2, 1, 3)
    xt = xt3[..., 0].reshape(N)
    yt = xt3[..., 1].reshape(N)
    zt = xt3[..., 2].reshape(N)

    p1 = pl.kernel(
        _p1_body,
        out_type=jax.ShapeDtypeStruct((NW, 8, L), i32),
        mesh=mesh,
        scratch_types=[
            pltpu.VMEM((WL * L,), f32),
            pltpu.VMEM((WL * L,), f32),
            pltpu.VMEM((WL * L,), f32),
            pltpu.VMEM((8, L), i32),
            pltpu.SemaphoreType.DMA,
            pltpu.SemaphoreType.DMA,
            pltpu.SemaphoreType.DMA,
        ],
    )
    hist = p1(xt, yt, zt)

    cnt = hist.transpose(0, 2, 1).reshape(NWL, 8)
    used = jnp.sum(cnt, axis=0)
    bpad = ((used + BTC - 1) // BTC) * BTC
    g = jnp.concatenate(
        [jnp.zeros((1,), i32), jnp.cumsum(bpad)]).astype(i32)
    lanebase = g[:8][None, :] + jnp.cumsum(cnt, axis=0) - cnt
    wbase = lanebase.reshape(NW, L, 8).transpose(0, 2, 1)
    jblk = jnp.arange(NB, dtype=i32) * BTC
    emap = jnp.clip(
        jnp.sum((jblk[:, None] >= g[1:9][None, :]).astype(i32), axis=1), 0, 7)
    vcnt = jnp.clip(used[emap] - (jblk - g[emap]), 0, BTC).astype(i32)
    vblk = jnp.concatenate([vcnt, jnp.zeros((NBP - NB,), i32)])

    p2 = pl.kernel(
        _p2_body,
        out_type=(
            jax.ShapeDtypeStruct((NPAD,), i32),
            jax.ShapeDtypeStruct((NPAD,), f32),
            jax.ShapeDtypeStruct((NPAD,), f32),
            jax.ShapeDtypeStruct((NPAD,), f32),
        ),
        mesh=mesh,
        scratch_types=[
            pltpu.VMEM((WL * L,), f32),
            pltpu.VMEM((WL * L,), f32),
            pltpu.VMEM((WL * L,), f32),
            pltpu.VMEM((WL * L,), i32),
            pltpu.VMEM((WL * L,), i32),
            pltpu.VMEM((8, L), i32),
            pltpu.SemaphoreType.DMA,
            pltpu.SemaphoreType.DMA,
            pltpu.SemaphoreType.DMA,
            pltpu.SemaphoreType.DMA,
            pltpu.SemaphoreType.DMA,
            pltpu.SemaphoreType.DMA,
            pltpu.SemaphoreType.DMA,
        ],
    )
    si, xs, ys, zs = p2(xt, yt, zt, wbase.astype(i32))

    span = local_max - local_min
    a = 2.0 / span
    c = -1.0 - 2.0 * local_min / span
    w1p = a[:, :, None] * W1
    b1p = jnp.einsum('ed,edh->eh', c, W1) + b1
    wl = jnp.zeros((8, 64, 8), f32)
    wl = wl.at[:, :, 0:3].set(jnp.transpose(w1p, (0, 2, 1)))
    wl = wl.at[:, :, 3].set(b1p)
    wl = wl.at[:, :, 4].set(b2)
    wl = wl.at[:, :, 5].set(W3[:, :, 0])
    wl = wl.at[:, :, 6].set(jnp.broadcast_to(b3, (8, 64)))
    w2t = jnp.transpose(W2, (0, 2, 1)).astype(jnp.bfloat16)
    emap3 = jnp.broadcast_to(emap[:, None, None], (NB, 1, 16))

    ysort = pl.pallas_call(
        _tc_body,
        grid=(NB,),
        in_specs=[
            pl.BlockSpec((1, 1, 16), lambda j: (j, 0, 0)),
            pl.BlockSpec((1, 1, BTC), lambda j: (j, 0, 0)),
            pl.BlockSpec((1, 1, BTC), lambda j: (j, 0, 0)),
            pl.BlockSpec((1, 1, BTC), lambda j: (j, 0, 0)),
            pl.BlockSpec((8, 64, 8), lambda j: (0, 0, 0)),
            pl.BlockSpec((8, 64, 64), lambda j: (0, 0, 0)),
        ],
        out_specs=pl.BlockSpec((1, 1, BTC), lambda j: (j, 0, 0)),
        out_shape=jax.ShapeDtypeStruct((NB, 1, BTC), f32),
    )(emap3, xs.reshape(NB, 1, BTC), ys.reshape(NB, 1, BTC),
      zs.reshape(NB, 1, BTC), wl, w2t)

    p3 = pl.kernel(
        _p3_body,
        out_type=jax.ShapeDtypeStruct((N + 16,), f32),
        mesh=mesh,
        scratch_types=[
            pltpu.VMEM((W3N,), f32),
            pltpu.VMEM((W3N,), i32),
            pltpu.VMEM((W3N,), i32),
            pltpu.VMEM((NBP,), i32),
            pltpu.SemaphoreType.DMA,
        ],
    )
    yfull = p3(ysort.reshape(NPAD), si, vblk)
    return yfull[:N].reshape(N, 1)


# speedup vs baseline: 4.7364x; 4.7364x over previous
import jax
import jax.numpy as jnp
from jax.experimental import pallas as pl
from jax.experimental.pallas import tpu as pltpu

E = 8
H = 64
B = 2048


def _mlp_block_kernel(x_ref, w1s_ref, b1s_ref, w2s_ref, wsm_ref, ones_ref,
                      out_ref):
    xb = x_ref[...]
    cell = (jnp.clip((xb + 1.0) * 0.5, 0.0, 0.99) * 2.0).astype(jnp.int32)
    flat = (cell[:, 0:1] + 2 * cell[:, 1:2] + 4 * cell[:, 2:3])

    xh = xb.astype(jnp.bfloat16)
    xl = (xb - xh.astype(jnp.float32)).astype(jnp.bfloat16)
    x6 = jnp.concatenate([xh, xl], axis=1)
    h1 = jnp.dot(x6, w1s_ref[...],
                 preferred_element_type=jnp.float32) + b1s_ref[...]

    col = jax.lax.broadcasted_iota(jnp.int32, (xb.shape[0], E * H), 1)
    keep = ((col // H) == flat) & (h1 > 0.0)
    a1 = jnp.where(keep, h1, 0.0).astype(jnp.bfloat16)

    col8 = jax.lax.broadcasted_iota(jnp.int32, (xb.shape[0], E), 1)
    onehot = (col8 == flat).astype(jnp.bfloat16)
    sm = jnp.dot(onehot, wsm_ref[...],
                 preferred_element_type=jnp.float32)

    h2 = jnp.maximum(
        jnp.dot(a1, w2s_ref[...],
                preferred_element_type=jnp.float32) + sm[:, :H], 0.0)

    prod = (h2 * sm[:, H:2 * H]).astype(jnp.bfloat16)
    y = jnp.dot(prod, ones_ref[...],
                preferred_element_type=jnp.float32) + sm[:, 2 * H:2 * H + 1]
    out_ref[...] = y


@jax.jit
def kernel(x, W1, b1, W2, b2, W3, b3, local_min, local_max):
    n = x.shape[0]
    span = local_max - local_min
    a = 2.0 / span
    c = -1.0 - 2.0 * local_min / span
    w1p = a[:, :, None] * W1
    b1p = jnp.einsum('ed,edh->eh', c, W1) + b1
    w1s = jnp.transpose(w1p, (1, 0, 2)).reshape(3, E * H)
    w1s6 = jnp.concatenate([w1s, w1s], axis=0).astype(jnp.bfloat16)
    b1s = b1p.reshape(1, E * H)
    w2s = W2.reshape(E * H, H).astype(jnp.bfloat16)
    wsm = jnp.concatenate([b2, W3[:, :, 0], b3], axis=1).astype(jnp.bfloat16)
    ones = jnp.ones((H, 1), jnp.bfloat16)

    grid = (n // B,)
    out = pl.pallas_call(
        _mlp_block_kernel,
        grid=grid,
        in_specs=[
            pl.BlockSpec((B, 3), lambda i: (i, 0)),
            pl.BlockSpec((6, E * H), lambda i: (0, 0)),
            pl.BlockSpec((1, E * H), lambda i: (0, 0)),
            pl.BlockSpec((E * H, H), lambda i: (0, 0)),
            pl.BlockSpec((E, 2 * H + 1), lambda i: (0, 0)),
            pl.BlockSpec((H, 1), lambda i: (0, 0)),
        ],
        out_specs=pl.BlockSpec((B, 1), lambda i: (i, 0)),
        out_shape=jax.ShapeDtypeStruct((n, 1), jnp.float32),
        compiler_params=pltpu.CompilerParams(
            dimension_semantics=("arbitrary",)),
    )(x, w1s6, b1s, w2s, wsm, ones)
    return out
